# baseline (device time: 177310 ns/iter reference)
import jax
import jax.numpy as jnp
from jax import lax
from jax.experimental import pallas as pl
from jax.experimental.pallas import tpu as pltpu

N_DEV = 4


def kernel(A, B):
    m, k = A.shape
    _, n = B.shape

    def body(a_ref, b_ref, out_ref, comm_ref, send_sems, recv_sems):
        my = lax.axis_index("i")
        left = (my - 1) % N_DEV
        right = (my + 1) % N_DEV

        barrier_sem = pltpu.get_barrier_semaphore()
        for nbr in [left, right]:
            pl.semaphore_signal(
                barrier_sem, inc=1,
                device_id=(nbr,), device_id_type=pl.DeviceIdType.MESH,
            )
        pl.semaphore_wait(barrier_sem, 2)

        partial = jnp.dot(
            a_ref[:, :].astype(jnp.bfloat16),
            b_ref[:, :].astype(jnp.bfloat16),
            preferred_element_type=jnp.float32,
        )
        out_ref[:, :] = partial
        comm_ref[0, :, :] = partial.astype(jnp.bfloat16)

        for h in range(N_DEV - 1):
            rdma = pltpu.make_async_remote_copy(
                src_ref=comm_ref.at[h],
                dst_ref=comm_ref.at[h + 1],
                send_sem=send_sems.at[h],
                recv_sem=recv_sems.at[h],
                device_id=(right,),
                device_id_type=pl.DeviceIdType.MESH,
            )
            rdma.start()
            rdma.wait()
            out_ref[:, :] = out_ref[:, :] + comm_ref[h + 1, :, :].astype(
                jnp.float32
            )

        z = out_ref[:, :]
        out_ref[:, :] = z / (1.0 + jnp.exp(-z))

    return pl.pallas_call(
        body,
        out_shape=jax.ShapeDtypeStruct((m, n), jnp.float32),
        in_specs=[
            pl.BlockSpec(memory_space=pltpu.VMEM),
            pl.BlockSpec(memory_space=pltpu.VMEM),
        ],
        out_specs=pl.BlockSpec(memory_space=pltpu.VMEM),
        scratch_shapes=[
            pltpu.VMEM((N_DEV, m, n), jnp.bfloat16),
            pltpu.SemaphoreType.DMA((N_DEV - 1,)),
            pltpu.SemaphoreType.DMA((N_DEV - 1,)),
        ],
        compiler_params=pltpu.CompilerParams(collective_id=0),
    )(A, B)


# device time: 65838 ns/iter; 2.6931x vs baseline; 2.6931x over previous
import jax
import jax.numpy as jnp
from jax import lax
from jax.experimental import pallas as pl
from jax.experimental.pallas import tpu as pltpu

N_DEV = 4


def kernel(A, B):
    m, k = A.shape
    _, n = B.shape
    half = m // 2
    ch = half // N_DEV

    def body(a_ref, b_ref, out_ref,
             rs_r, rs_l, ag_r, ag_l,
             rs_r_ssem, rs_r_rsem, rs_l_ssem, rs_l_rsem,
             ag_r_ssem, ag_r_rsem, ag_l_ssem, ag_l_rsem):
        my = lax.axis_index("i")
        left = (my - 1) % N_DEV
        right = (my + 1) % N_DEV

        barrier_sem = pltpu.get_barrier_semaphore()
        for nbr in [left, right]:
            pl.semaphore_signal(
                barrier_sem, inc=1,
                device_id=(nbr,), device_id_type=pl.DeviceIdType.MESH,
            )
        pl.semaphore_wait(barrier_sem, 2)

        out_ref[:, :] = jnp.dot(
            a_ref[:, :].astype(jnp.bfloat16),
            b_ref[:, :].astype(jnp.bfloat16),
            preferred_element_type=jnp.float32,
        )

        def top(c):
            return pl.ds(c * ch, ch)

        def bot(c):
            return pl.ds(half + c * ch, ch)

        rs_r[0, :, :] = out_ref[top(my), :].astype(jnp.bfloat16)
        rs_l[0, :, :] = out_ref[bot(my), :].astype(jnp.bfloat16)

        for h in range(N_DEV - 1):
            r_rdma = pltpu.make_async_remote_copy(
                src_ref=rs_r.at[h], dst_ref=rs_r.at[h + 1],
                send_sem=rs_r_ssem.at[h], recv_sem=rs_r_rsem.at[h],
                device_id=(right,), device_id_type=pl.DeviceIdType.MESH)
            l_rdma = pltpu.make_async_remote_copy(
                src_ref=rs_l.at[h], dst_ref=rs_l.at[h + 1],
                send_sem=rs_l_ssem.at[h], recv_sem=rs_l_rsem.at[h],
                device_id=(left,), device_id_type=pl.DeviceIdType.MESH)
            r_rdma.start()
            l_rdma.start()
            r_rdma.wait()
            l_rdma.wait()

            ctr = (my - h - 1) % N_DEV
            cbl = (my + h + 1) % N_DEV
            acc_r = out_ref[top(ctr), :] + rs_r[h + 1, :, :].astype(jnp.float32)
            acc_l = out_ref[bot(cbl), :] + rs_l[h + 1, :, :].astype(jnp.float32)
            if h < N_DEV - 2:
                rs_r[h + 1, :, :] = acc_r.astype(jnp.bfloat16)
                rs_l[h + 1, :, :] = acc_l.astype(jnp.bfloat16)
            else:
                silu_r = acc_r / (1.0 + jnp.exp(-acc_r))
                silu_l = acc_l / (1.0 + jnp.exp(-acc_l))
                out_ref[top(ctr), :] = silu_r
                out_ref[bot(cbl), :] = silu_l
                ag_r[0, :, :] = silu_r.astype(jnp.bfloat16)
                ag_l[0, :, :] = silu_l.astype(jnp.bfloat16)

        for h in range(N_DEV - 1):
            r_rdma = pltpu.make_async_remote_copy(
                src_ref=ag_r.at[h], dst_ref=ag_r.at[h + 1],
                send_sem=ag_r_ssem.at[h], recv_sem=ag_r_rsem.at[h],
                device_id=(right,), device_id_type=pl.DeviceIdType.MESH)
            l_rdma = pltpu.make_async_remote_copy(
                src_ref=ag_l.at[h], dst_ref=ag_l.at[h + 1],
                send_sem=ag_l_ssem.at[h], recv_sem=ag_l_rsem.at[h],
                device_id=(left,), device_id_type=pl.DeviceIdType.MESH)
            r_rdma.start()
            l_rdma.start()
            r_rdma.wait()
            l_rdma.wait()

            ctr = (my - h) % N_DEV
            cbl = (my + h) % N_DEV
            out_ref[top(ctr), :] = ag_r[h + 1, :, :].astype(jnp.float32)
            out_ref[bot(cbl), :] = ag_l[h + 1, :, :].astype(jnp.float32)

    return pl.pallas_call(
        body,
        out_shape=jax.ShapeDtypeStruct((m, n), jnp.float32),
        in_specs=[
            pl.BlockSpec(memory_space=pltpu.VMEM),
            pl.BlockSpec(memory_space=pltpu.VMEM),
        ],
        out_specs=pl.BlockSpec(memory_space=pltpu.VMEM),
        scratch_shapes=[
            pltpu.VMEM((N_DEV, ch, n), jnp.bfloat16),
            pltpu.VMEM((N_DEV, ch, n), jnp.bfloat16),
            pltpu.VMEM((N_DEV, ch, n), jnp.bfloat16),
            pltpu.VMEM((N_DEV, ch, n), jnp.bfloat16),
            pltpu.SemaphoreType.DMA((N_DEV - 1,)),
            pltpu.SemaphoreType.DMA((N_DEV - 1,)),
            pltpu.SemaphoreType.DMA((N_DEV - 1,)),
            pltpu.SemaphoreType.DMA((N_DEV - 1,)),
            pltpu.SemaphoreType.DMA((N_DEV - 1,)),
            pltpu.SemaphoreType.DMA((N_DEV - 1,)),
            pltpu.SemaphoreType.DMA((N_DEV - 1,)),
            pltpu.SemaphoreType.DMA((N_DEV - 1,)),
        ],
        compiler_params=pltpu.CompilerParams(collective_id=0),
    )(A, B)


# device time: 65245 ns/iter; 2.7176x vs baseline; 1.0091x over previous
import jax
import jax.numpy as jnp
from jax import lax
from jax.experimental import pallas as pl
from jax.experimental.pallas import tpu as pltpu

N_DEV = 4


def kernel(A, B):
    m, k = A.shape
    _, n = B.shape
    half = m // 2
    ch = half // N_DEV

    def body(a_ref, b_ref, out_ref,
             rs_r, rs_l, ag_r, ag_l,
             rs_r_ssem, rs_r_rsem, rs_l_ssem, rs_l_rsem,
             ag_r_ssem, ag_r_rsem, ag_l_ssem, ag_l_rsem):
        my = lax.axis_index("i")
        left = (my - 1) % N_DEV
        right = (my + 1) % N_DEV

        barrier_sem = pltpu.get_barrier_semaphore()
        for nbr in [left, right]:
            pl.semaphore_signal(
                barrier_sem, inc=1,
                device_id=(nbr,), device_id_type=pl.DeviceIdType.MESH,
            )
        pl.semaphore_wait(barrier_sem, 2)

        b16 = b_ref[:, :].astype(jnp.bfloat16)

        def top(c):
            return pl.ds(c * ch, ch)

        def bot(c):
            return pl.ds(half + c * ch, ch)

        def pdot(rows):
            return jnp.dot(
                a_ref[rows, :].astype(jnp.bfloat16), b16,
                preferred_element_type=jnp.float32,
            )

        def mk(buf, ssem, rsem, h, dev):
            return pltpu.make_async_remote_copy(
                src_ref=buf.at[h], dst_ref=buf.at[h + 1],
                send_sem=ssem.at[h], recv_sem=rsem.at[h],
                device_id=(dev,), device_id_type=pl.DeviceIdType.MESH)

        def silu(v):
            return v / (1.0 + jnp.exp(-v))

        rs_r[0, :, :] = pdot(top(my)).astype(jnp.bfloat16)
        rs_l[0, :, :] = pdot(bot(my)).astype(jnp.bfloat16)
        rs_rd = [(mk(rs_r, rs_r_ssem, rs_r_rsem, 0, right),
                  mk(rs_l, rs_l_ssem, rs_l_rsem, 0, left))]
        rs_rd[0][0].start()
        rs_rd[0][1].start()

        ag_rd = []
        for h in range(N_DEV - 1):
            ctr = (my - h - 1) % N_DEV
            cbl = (my + h + 1) % N_DEV
            p_t = pdot(top(ctr))
            p_b = pdot(bot(cbl))
            rr, rl = rs_rd[h]
            rr.wait()
            rl.wait()
            acc_r = p_t + rs_r[h + 1, :, :].astype(jnp.float32)
            acc_l = p_b + rs_l[h + 1, :, :].astype(jnp.float32)
            if h < N_DEV - 2:
                rs_r[h + 1, :, :] = acc_r.astype(jnp.bfloat16)
                rs_l[h + 1, :, :] = acc_l.astype(jnp.bfloat16)
                nxt = (mk(rs_r, rs_r_ssem, rs_r_rsem, h + 1, right),
                       mk(rs_l, rs_l_ssem, rs_l_rsem, h + 1, left))
                nxt[0].start()
                nxt[1].start()
                rs_rd.append(nxt)
            else:
                ag_r[0, :, :] = acc_r.astype(jnp.bfloat16)
                ag_l[0, :, :] = acc_l.astype(jnp.bfloat16)
                ag_rd.append((mk(ag_r, ag_r_ssem, ag_r_rsem, 0, right),
                              mk(ag_l, ag_l_ssem, ag_l_rsem, 0, left)))
                ag_rd[0][0].start()
                ag_rd[0][1].start()
                out_ref[top(ctr), :] = silu(acc_r)
                out_ref[bot(cbl), :] = silu(acc_l)

        for h in range(N_DEV - 1):
            ar, al = ag_rd[h]
            ar.wait()
            al.wait()
            if h < N_DEV - 2:
                nxt = (mk(ag_r, ag_r_ssem, ag_r_rsem, h + 1, right),
                       mk(ag_l, ag_l_ssem, ag_l_rsem, h + 1, left))
                nxt[0].start()
                nxt[1].start()
                ag_rd.append(nxt)
            ctr = (my - h) % N_DEV
            cbl = (my + h) % N_DEV
            out_ref[top(ctr), :] = silu(ag_r[h + 1, :, :].astype(jnp.float32))
            out_ref[bot(cbl), :] = silu(ag_l[h + 1, :, :].astype(jnp.float32))

    return pl.pallas_call(
        body,
        out_shape=jax.ShapeDtypeStruct((m, n), jnp.float32),
        in_specs=[
            pl.BlockSpec(memory_space=pltpu.VMEM),
            pl.BlockSpec(memory_space=pltpu.VMEM),
        ],
        out_specs=pl.BlockSpec(memory_space=pltpu.VMEM),
        scratch_shapes=[
            pltpu.VMEM((N_DEV, ch, n), jnp.bfloat16),
            pltpu.VMEM((N_DEV, ch, n), jnp.bfloat16),
            pltpu.VMEM((N_DEV, ch, n), jnp.bfloat16),
            pltpu.VMEM((N_DEV, ch, n), jnp.bfloat16),
            pltpu.SemaphoreType.DMA((N_DEV - 1,)),
            pltpu.SemaphoreType.DMA((N_DEV - 1,)),
            pltpu.SemaphoreType.DMA((N_DEV - 1,)),
            pltpu.SemaphoreType.DMA((N_DEV - 1,)),
            pltpu.SemaphoreType.DMA((N_DEV - 1,)),
            pltpu.SemaphoreType.DMA((N_DEV - 1,)),
            pltpu.SemaphoreType.DMA((N_DEV - 1,)),
            pltpu.SemaphoreType.DMA((N_DEV - 1,)),
        ],
        compiler_params=pltpu.CompilerParams(collective_id=0),
    )(A, B)
